# baseline (speedup 1.0000x reference)
import functools

import jax
import jax.numpy as jnp
from jax import lax
from jax.experimental import pallas as pl
from jax.experimental.pallas import tpu as pltpu
from jax.experimental.pallas import tpu_sc as plsc

_FEAT_DIM = 512
_CATE_NUM = 1000
_SHOTS_TOTAL = 32
_FEAT_NUM = 256
_ROWS = _CATE_NUM * _SHOTS_TOTAL

_BLK_CLS = 40

_NC, _NS = 2, 16
_NW = _NC * _NS
_ROWS_W = _ROWS // _NW
_CHUNK = 40
_NCH = _ROWS_W // _CHUNK
_VCOLS = 63


def _tc_body(idx_ref, res_full_ref, cw_ref, ck_ref, nck_ref, ncw_ref, res_exp_ref):
    i = pl.program_id(0)

    @pl.when(i == 0)
    def _():
        col = jax.lax.broadcasted_iota(jnp.int32, (_FEAT_NUM, _FEAT_DIM), 1)
        s = (idx_ref[...] == col).astype(jnp.float32)
        res_exp_ref[...] = jnp.dot(res_full_ref[...], s,
                                   preferred_element_type=jnp.float32)
        ncw_ref[...] = cw_ref[...] + jax.lax.dot_general(
            s, res_full_ref[...], (((0,), (1,)), ((), ())),
            preferred_element_type=jnp.float32)

    add = res_exp_ref[pl.ds(i * _BLK_CLS, _BLK_CLS), :]
    rep = jnp.broadcast_to(add[:, None, :], (_BLK_CLS, _SHOTS_TOTAL, _FEAT_DIM))
    nck_ref[...] = ck_ref[...] + rep.reshape(_BLK_CLS * _SHOTS_TOTAL, _FEAT_DIM)


_sc_mesh = plsc.VectorSubcoreMesh(core_axis_name="c", subcore_axis_name="s",
                                  num_cores=_NC, num_subcores=_NS)


@functools.partial(
    pl.kernel,
    out_type=jax.ShapeDtypeStruct((_ROWS, _CATE_NUM), jnp.float32),
    mesh=_sc_mesh,
    scratch_types=[
        pltpu.VMEM((_CHUNK, _CATE_NUM), jnp.float32),
        pltpu.VMEM((_CHUNK, _CATE_NUM), jnp.float32),
        pltpu.VMEM((_ROWS_W * 16,), jnp.float32),
        pltpu.SemaphoreType.DMA,
        pltpu.SemaphoreType.DMA,
        pltpu.SemaphoreType.DMA,
        pltpu.SemaphoreType.DMA,
    ],
)
def _sc_cv_kernel(cv_hbm, vw_hbm, ncv_hbm, buf0, buf1, vwbuf,
                  sin0, sin1, sout0, sout1):
    w = lax.axis_index("s") * _NC + lax.axis_index("c")
    base = w * _ROWS_W
    pltpu.sync_copy(vw_hbm.at[pl.ds(base * 16, _ROWS_W * 16)], vwbuf)
    bufs = (buf0, buf1)
    sins = (sin0, sin1)
    souts = (sout0, sout1)

    def g_start(c, b):
        pltpu.async_copy(cv_hbm.at[pl.ds(base + c * _CHUNK, _CHUNK)],
                         bufs[b], sins[b])

    def g_wait(c, b):
        pltpu.make_async_copy(cv_hbm.at[pl.ds(base + c * _CHUNK, _CHUNK)],
                              bufs[b], sins[b]).wait()

    def s_start(c, b):
        pltpu.async_copy(bufs[b], ncv_hbm.at[pl.ds(base + c * _CHUNK, _CHUNK)],
                         souts[b])

    def s_wait(c, b):
        pltpu.make_async_copy(bufs[b], ncv_hbm.at[pl.ds(base + c * _CHUNK, _CHUNK)],
                              souts[b]).wait()

    g_start(0, 0)

    def chunk_fn(c, b):
        @pl.when(c >= 1)
        def _():
            s_wait(c - 1, 1 - b)

        @pl.when(c + 1 < _NCH)
        def _():
            g_start(c + 1, 1 - b)

        g_wait(c, b)
        buf = bufs[b]

        def row_fn(r, carry):
            splat = vwbuf[pl.ds(pl.multiple_of((c * _CHUNK + r) * 16, 16), 16)]
            for o in range(_VCOLS):
                off = 984 if o == _VCOLS - 1 else o * 16
                buf[r, pl.ds(off, 16)] = buf[r, pl.ds(off, 16)] * splat
            return carry

        lax.fori_loop(0, _CHUNK, row_fn, 0)
        s_start(c, b)

    def outer(i, carry):
        chunk_fn(2 * i, 0)
        chunk_fn(2 * i + 1, 1)
        return carry

    lax.fori_loop(0, _NCH // 2, outer, 0)
    if _NCH % 2:
        chunk_fn(jnp.int32(_NCH - 1), 0)
        s_wait(_NCH - 1, 0)
    else:
        s_wait(_NCH - 1, 1)


def kernel(cache_keys, clip_weights, cache_values, res, value_weights, indices):
    idx = indices.astype(jnp.int32).reshape(_FEAT_NUM, 1)
    blk_rows = _BLK_CLS * _SHOTS_TOTAL
    grid = _CATE_NUM // _BLK_CLS
    nck, ncw = pl.pallas_call(
        _tc_body,
        grid=(grid,),
        in_specs=[
            pl.BlockSpec((_FEAT_NUM, 1), lambda i: (0, 0)),
            pl.BlockSpec((_CATE_NUM, _FEAT_NUM), lambda i: (0, 0)),
            pl.BlockSpec((_FEAT_DIM, _CATE_NUM), lambda i: (0, 0)),
            pl.BlockSpec((blk_rows, _FEAT_DIM), lambda i: (i, 0)),
        ],
        out_specs=[
            pl.BlockSpec((blk_rows, _FEAT_DIM), lambda i: (i, 0)),
            pl.BlockSpec((_FEAT_DIM, _CATE_NUM), lambda i: (0, 0)),
        ],
        out_shape=[
            jax.ShapeDtypeStruct((_ROWS, _FEAT_DIM), jnp.float32),
            jax.ShapeDtypeStruct((_FEAT_DIM, _CATE_NUM), jnp.float32),
        ],
        scratch_shapes=[pltpu.VMEM((_CATE_NUM, _FEAT_DIM), jnp.float32)],
    )(idx, res, clip_weights, cache_keys)
    vwx = jnp.broadcast_to(value_weights.reshape(_ROWS, 1),
                           (_ROWS, 16)).reshape(_ROWS * 16)
    ncv = _sc_cv_kernel(cache_values, vwx)
    return (nck, ncw, ncv)
